# baseline (speedup 1.0000x reference)
import functools
import jax
import jax.numpy as jnp
from jax import lax
from jax.experimental import pallas as pl
from jax.experimental.pallas import tpu as pltpu
from jax.experimental.pallas import tpu_sc as plsc

E = 320000
D = 128
LANES = 16
NCH = D // LANES
NW = 32
EPW = E // NW
BLK = 400
NBLK = EPW // BLK
SUB = 80
NSUB = BLK // SUB
NTAB = 70
FB = 4 * BLK


def _body(feat_hbm, bond_hbm, st_hbm, wt_hbm, b_hbm, out_hbm,
          c0x, c1x, c2x, c3x, idx2, rows2, bond_v, st_v, wt_v, b_v,
          tab_v, tab_sh, gsem, fsem, osem):
    sid = lax.axis_index("s")
    wid = sid * 2 + lax.axis_index("c")
    base = wid * EPW

    pltpu.sync_copy(bond_hbm, bond_v)
    pltpu.sync_copy(st_hbm, st_v)
    pltpu.sync_copy(wt_hbm, wt_v)
    pltpu.sync_copy(b_hbm, b_v)

    def build_row(i, _):
        bi = i // 7
        si = i - bi * 7
        for c in range(NCH):
            tab_v[i, pl.ds(c * LANES, LANES)] = (
                bond_v[pl.ds(bi * D + c * LANES, LANES)]
                + st_v[pl.ds(si * D + c * LANES, LANES)]
                + b_v[pl.ds(c * LANES, LANES)]
            )
        return 0

    lax.fori_loop(0, NTAB, build_row, 0)

    pltpu.sync_copy(tab_v, tab_sh.at[pl.ds(sid * NTAB, NTAB), :])

    w0 = [wt_v[pl.ds(c * LANES, LANES)] for c in range(NCH)]
    w1 = [wt_v[pl.ds(D + c * LANES, LANES)] for c in range(NCH)]

    def feat_start(j):
        eb = base + j * BLK
        p = j & 1
        q = j - (j // 3) * 3
        pltpu.async_copy(feat_hbm.at[pl.ds(0 * E + eb, BLK)],
                         c0x.at[pl.ds(p * BLK, BLK)], fsem)
        pltpu.async_copy(feat_hbm.at[pl.ds(1 * E + eb, BLK)],
                         c1x.at[pl.ds(q * BLK, BLK)], fsem)
        pltpu.async_copy(feat_hbm.at[pl.ds(2 * E + eb, BLK)],
                         c2x.at[pl.ds(q * BLK, BLK)], fsem)
        pltpu.async_copy(feat_hbm.at[pl.ds(3 * E + eb, BLK)],
                         c3x.at[pl.ds(p * BLK, BLK)], fsem)

    def feat_wait():
        for r in (c0x, c1x, c2x, c3x):
            pltpu.make_async_copy(feat_hbm.at[pl.ds(0, BLK)],
                                  r.at[pl.ds(0, BLK)], fsem).wait()

    def idx_compute(j):
        p = j & 1

        @plsc.parallel_loop(0, BLK // 16, unroll=2)
        def grp(g):
            f0 = c0x[pl.ds(p * BLK + g * 16, 16)]
            f3 = c3x[pl.ds(p * BLK + g * 16, 16)]
            bond = jnp.clip((f0 * 2.0).astype(jnp.int32), 0, 9)
            st = jnp.clip(f3.astype(jnp.int32), 0, 6)
            r = g // (SUB // 16)
            co = (g - r * (SUB // 16)) * 16
            idx2[p * NSUB + r, pl.ds(co, 16)] = bond * 7 + st + sid * NTAB

    def gather_start(j):
        p = j & 1
        for i in range(NSUB):
            pltpu.async_copy(tab_sh.at[idx2.at[p * NSUB + i]],
                             rows2.at[pl.ds(p * BLK + i * SUB, SUB), :], gsem)

    def gather_wait_n(k):
        pltpu.make_async_copy(tab_sh.at[idx2.at[0]],
                              rows2.at[pl.ds(0, k * SUB), :], gsem).wait()

    def out_start(j):
        p = j & 1
        eb = base + j * BLK
        pltpu.async_copy(rows2.at[pl.ds(p * BLK, BLK), :],
                         out_hbm.at[pl.ds(eb, BLK), :], osem)

    def out_wait():
        pltpu.make_async_copy(rows2.at[pl.ds(0, BLK), :],
                              out_hbm.at[pl.ds(base, BLK), :], osem).wait()

    feat_start(0)
    feat_start(1)
    feat_wait()
    idx_compute(0)
    gather_start(0)

    def block_body(j, _):
        p = j & 1
        q = j - (j // 3) * 3

        @pl.when(j < NBLK - 2)
        def _():
            feat_start(j + 2)

        @pl.when(j < NBLK - 1)
        def _():
            feat_wait()
            idx_compute(j + 1)

        def lin_part(lo, hi):
            @plsc.parallel_loop(lo, hi, unroll=1)
            def lin(g):
                cj = c1x[pl.ds(q * BLK + g * 16, 16)]
                rg = c2x[pl.ds(q * BLK + g * 16, 16)]
                rb = p * BLK + g * 16
                for k in range(16):
                    cjk = cj[k]
                    rgk = rg[k]
                    for c in range(NCH):
                        plsc.addupdate(
                            rows2.at[rb + k, pl.ds(c * LANES, LANES)],
                            cjk * w0[c] + rgk * w1[c],
                        )

        for i in range(3):
            gather_wait_n(1)
            lin_part(5 * i, 5 * i + 5)

        @pl.when(j >= 1)
        def _():
            out_wait()

        @pl.when(j < NBLK - 1)
        def _():
            gather_start(j + 1)

        for i in range(3, 5):
            gather_wait_n(1)
            lin_part(5 * i, 5 * i + 5)

        out_start(j)
        return 0

    lax.fori_loop(0, NBLK, block_body, 0)
    out_wait()


@jax.jit
def _run(feat, bond, st, wt, b):
    mesh = plsc.VectorSubcoreMesh(core_axis_name="c", subcore_axis_name="s")
    return pl.kernel(
        _body,
        out_type=jax.ShapeDtypeStruct((E, D), jnp.float32),
        mesh=mesh,
        scratch_types=[
            pltpu.VMEM((2 * BLK,), jnp.float32),
            pltpu.VMEM((3 * BLK,), jnp.float32),
            pltpu.VMEM((3 * BLK,), jnp.float32),
            pltpu.VMEM((2 * BLK,), jnp.float32),
            pltpu.VMEM((2 * NSUB, SUB), jnp.int32),
            pltpu.VMEM((2 * BLK, D), jnp.float32),
            pltpu.VMEM((10 * D,), jnp.float32),
            pltpu.VMEM((7 * D,), jnp.float32),
            pltpu.VMEM((2 * D,), jnp.float32),
            pltpu.VMEM((D,), jnp.float32),
            pltpu.VMEM((NTAB, D), jnp.float32),
            pltpu.VMEM_SHARED((16 * NTAB, D), jnp.float32),
            pltpu.SemaphoreType.DMA,
            pltpu.SemaphoreType.DMA,
            pltpu.SemaphoreType.DMA,
        ],
    )(feat, bond, st, wt, b)


def kernel(edge_features, bond_type_table, stereo_table, W_binary, b_binary):
    feat = edge_features.T.reshape(-1)
    bond = bond_type_table.reshape(-1)
    st = stereo_table.reshape(-1)
    wt = W_binary.T.reshape(-1)
    return _run(feat, bond, st, wt, b_binary)


# speedup vs baseline: 1.1251x; 1.1251x over previous
import functools
import jax
import jax.numpy as jnp
from jax import lax
from jax.experimental import pallas as pl
from jax.experimental.pallas import tpu as pltpu
from jax.experimental.pallas import tpu_sc as plsc

E = 320000
D = 128
LANES = 16
NCH = D // LANES
NW = 32
EPW = E // NW
BLK = 400
NBLK = EPW // BLK
SUB = 80
NSUB = BLK // SUB
NTAB = 70
FB = 4 * BLK


def _body(feat_hbm, bond_hbm, st_hbm, wt_hbm, b_hbm, out_hbm,
          c0x, c1x, c2x, c3x, idx2, rows2, bond_v, st_v, wt_v, b_v,
          tab_v, tab_sh, gsem, fsem, osem):
    sid = lax.axis_index("s")
    wid = sid * 2 + lax.axis_index("c")
    base = wid * EPW

    pltpu.sync_copy(bond_hbm, bond_v)
    pltpu.sync_copy(st_hbm, st_v)
    pltpu.sync_copy(wt_hbm, wt_v)
    pltpu.sync_copy(b_hbm, b_v)

    def build_row(i, _):
        bi = i // 7
        si = i - bi * 7
        for c in range(NCH):
            tab_v[i, pl.ds(c * LANES, LANES)] = (
                bond_v[pl.ds(bi * D + c * LANES, LANES)]
                + st_v[pl.ds(si * D + c * LANES, LANES)]
                + b_v[pl.ds(c * LANES, LANES)]
            )
        return 0

    lax.fori_loop(0, NTAB, build_row, 0)

    pltpu.sync_copy(tab_v, tab_sh.at[pl.ds(sid * NTAB, NTAB), :])

    w0 = [wt_v[pl.ds(c * LANES, LANES)] for c in range(NCH)]
    w1 = [wt_v[pl.ds(D + c * LANES, LANES)] for c in range(NCH)]

    def feat_start(j):
        eb = base + j * BLK
        p = j & 1
        q = j - (j // 3) * 3
        pltpu.async_copy(feat_hbm.at[pl.ds(0 * E + eb, BLK)],
                         c0x.at[pl.ds(p * BLK, BLK)], fsem)
        pltpu.async_copy(feat_hbm.at[pl.ds(1 * E + eb, BLK)],
                         c1x.at[pl.ds(q * BLK, BLK)], fsem)
        pltpu.async_copy(feat_hbm.at[pl.ds(2 * E + eb, BLK)],
                         c2x.at[pl.ds(q * BLK, BLK)], fsem)
        pltpu.async_copy(feat_hbm.at[pl.ds(3 * E + eb, BLK)],
                         c3x.at[pl.ds(p * BLK, BLK)], fsem)

    def feat_wait():
        for r in (c0x, c1x, c2x, c3x):
            pltpu.make_async_copy(feat_hbm.at[pl.ds(0, BLK)],
                                  r.at[pl.ds(0, BLK)], fsem).wait()

    def idx_compute(j):
        p = j & 1

        @plsc.parallel_loop(0, BLK // 16, unroll=2)
        def grp(g):
            f0 = c0x[pl.ds(p * BLK + g * 16, 16)]
            f3 = c3x[pl.ds(p * BLK + g * 16, 16)]
            bond = jnp.clip((f0 * 2.0).astype(jnp.int32), 0, 9)
            st = jnp.clip(f3.astype(jnp.int32), 0, 6)
            r = g // (SUB // 16)
            co = (g - r * (SUB // 16)) * 16
            idx2[p * NSUB + r, pl.ds(co, 16)] = bond * 7 + st + sid * NTAB

    def gather_start(j):
        p = j & 1
        for i in range(NSUB):
            pltpu.async_copy(tab_sh.at[idx2.at[p * NSUB + i]],
                             rows2.at[pl.ds(p * BLK + i * SUB, SUB), :], gsem)

    def gather_wait_n(k):
        pltpu.make_async_copy(tab_sh.at[idx2.at[0]],
                              rows2.at[pl.ds(0, k * SUB), :], gsem).wait()

    def out_start(j):
        p = j & 1
        eb = base + j * BLK
        pltpu.async_copy(rows2.at[pl.ds(p * BLK, BLK), :],
                         out_hbm.at[pl.ds(eb, BLK), :], osem)

    def out_wait():
        pltpu.make_async_copy(rows2.at[pl.ds(0, BLK), :],
                              out_hbm.at[pl.ds(base, BLK), :], osem).wait()

    feat_start(0)
    feat_start(1)
    feat_wait()
    idx_compute(0)
    gather_start(0)

    def block_body(j, _):
        p = j & 1
        q = j - (j // 3) * 3

        @pl.when(j < NBLK - 2)
        def _():
            feat_start(j + 2)

        @pl.when(j < NBLK - 1)
        def _():
            feat_wait()
            idx_compute(j + 1)

        def lin_part(lo, hi):
            @plsc.parallel_loop(lo, hi, unroll=1)
            def lin(g):
                cj = c1x[pl.ds(q * BLK + g * 16, 16)]
                rg = c2x[pl.ds(q * BLK + g * 16, 16)]
                rb = p * BLK + g * 16
                for k in range(16):
                    cjk = cj[k]
                    rgk = rg[k]
                    for c in range(NCH):
                        plsc.addupdate(
                            rows2.at[rb + k, pl.ds(c * LANES, LANES)],
                            cjk * w0[c] + rgk * w1[c],
                        )

        gather_wait_n(3)
        lin_part(0, 15)

        gather_wait_n(2)

        @pl.when(j >= 1)
        def _():
            out_wait()

        @pl.when(j < NBLK - 1)
        def _():
            gather_start(j + 1)

        lin_part(15, BLK // 16)

        out_start(j)
        return 0

    lax.fori_loop(0, NBLK, block_body, 0)
    out_wait()


@jax.jit
def _run(feat, bond, st, wt, b):
    mesh = plsc.VectorSubcoreMesh(core_axis_name="c", subcore_axis_name="s")
    return pl.kernel(
        _body,
        out_type=jax.ShapeDtypeStruct((E, D), jnp.float32),
        mesh=mesh,
        scratch_types=[
            pltpu.VMEM((2 * BLK,), jnp.float32),
            pltpu.VMEM((3 * BLK,), jnp.float32),
            pltpu.VMEM((3 * BLK,), jnp.float32),
            pltpu.VMEM((2 * BLK,), jnp.float32),
            pltpu.VMEM((2 * NSUB, SUB), jnp.int32),
            pltpu.VMEM((2 * BLK, D), jnp.float32),
            pltpu.VMEM((10 * D,), jnp.float32),
            pltpu.VMEM((7 * D,), jnp.float32),
            pltpu.VMEM((2 * D,), jnp.float32),
            pltpu.VMEM((D,), jnp.float32),
            pltpu.VMEM((NTAB, D), jnp.float32),
            pltpu.VMEM_SHARED((16 * NTAB, D), jnp.float32),
            pltpu.SemaphoreType.DMA,
            pltpu.SemaphoreType.DMA,
            pltpu.SemaphoreType.DMA,
        ],
    )(feat, bond, st, wt, b)


def kernel(edge_features, bond_type_table, stereo_table, W_binary, b_binary):
    feat = edge_features.T.reshape(-1)
    bond = bond_type_table.reshape(-1)
    st = stereo_table.reshape(-1)
    wt = W_binary.T.reshape(-1)
    return _run(feat, bond, st, wt, b_binary)


# speedup vs baseline: 1.1269x; 1.0015x over previous
import functools
import jax
import jax.numpy as jnp
from jax import lax
from jax.experimental import pallas as pl
from jax.experimental.pallas import tpu as pltpu
from jax.experimental.pallas import tpu_sc as plsc

E = 320000
D = 128
LANES = 16
NCH = D // LANES
NW = 32
EPW = E // NW
BLK = 400
NBLK = EPW // BLK
SUB = 80
NSUB = BLK // SUB
NTAB = 70
FB = 4 * BLK


def _body(feat_hbm, bond_hbm, st_hbm, wt_hbm, b_hbm, out_hbm,
          colsx, idx2, rows2, bond_v, st_v, wt_v, b_v,
          tab_v, tab_sh, gsem, fsem, osem):
    sid = lax.axis_index("s")
    wid = sid * 2 + lax.axis_index("c")
    base = wid * EPW

    pltpu.sync_copy(bond_hbm, bond_v)
    pltpu.sync_copy(st_hbm, st_v)
    pltpu.sync_copy(wt_hbm, wt_v)
    pltpu.sync_copy(b_hbm, b_v)

    def build_row(i, _):
        bi = i // 7
        si = i - bi * 7
        for c in range(NCH):
            tab_v[i, pl.ds(c * LANES, LANES)] = (
                bond_v[pl.ds(bi * D + c * LANES, LANES)]
                + st_v[pl.ds(si * D + c * LANES, LANES)]
                + b_v[pl.ds(c * LANES, LANES)]
            )
        return 0

    lax.fori_loop(0, NTAB, build_row, 0)

    pltpu.sync_copy(tab_v, tab_sh.at[pl.ds(sid * NTAB, NTAB), :])

    w0 = [wt_v[pl.ds(c * LANES, LANES)] for c in range(NCH)]
    w1 = [wt_v[pl.ds(D + c * LANES, LANES)] for c in range(NCH)]

    FB4 = 4 * BLK

    def feat_start(j):
        eb = base + j * BLK
        q = j - (j // 3) * 3
        for c in range(4):
            pltpu.async_copy(feat_hbm.at[pl.ds(c * E + eb, BLK)],
                             colsx.at[pl.ds(q * FB4 + c * BLK, BLK)], fsem)

    def feat_wait():
        pltpu.make_async_copy(feat_hbm.at[pl.ds(0, FB4)],
                              colsx.at[pl.ds(0, FB4)], fsem).wait()

    def idx_compute(j):
        p = j & 1
        qj = j - (j // 3) * 3

        @plsc.parallel_loop(0, BLK // 16, unroll=2)
        def grp(g):
            f0 = colsx[pl.ds(qj * 4 * BLK + 0 * BLK + g * 16, 16)]
            f3 = colsx[pl.ds(qj * 4 * BLK + 3 * BLK + g * 16, 16)]
            bond = jnp.clip((f0 * 2.0).astype(jnp.int32), 0, 9)
            st = jnp.clip(f3.astype(jnp.int32), 0, 6)
            r = g // (SUB // 16)
            co = (g - r * (SUB // 16)) * 16
            idx2[p * NSUB + r, pl.ds(co, 16)] = bond * 7 + st + sid * NTAB

    def gather_start(j):
        p = j & 1
        for i in range(NSUB):
            pltpu.async_copy(tab_sh.at[idx2.at[p * NSUB + i]],
                             rows2.at[pl.ds(p * BLK + i * SUB, SUB), :], gsem)

    def gather_wait_n(k):
        pltpu.make_async_copy(tab_sh.at[idx2.at[0]],
                              rows2.at[pl.ds(0, k * SUB), :], gsem).wait()

    def out_start(j):
        p = j & 1
        eb = base + j * BLK
        pltpu.async_copy(rows2.at[pl.ds(p * BLK, BLK), :],
                         out_hbm.at[pl.ds(eb, BLK), :], osem)

    def out_wait():
        pltpu.make_async_copy(rows2.at[pl.ds(0, BLK), :],
                              out_hbm.at[pl.ds(base, BLK), :], osem).wait()

    feat_start(0)
    feat_start(1)
    feat_wait()
    idx_compute(0)
    gather_start(0)

    def block_body(j, _):
        p = j & 1
        q = j - (j // 3) * 3

        @pl.when(j < NBLK - 2)
        def _():
            feat_start(j + 2)

        @pl.when(j < NBLK - 1)
        def _():
            feat_wait()
            idx_compute(j + 1)

        def lin_part(lo, hi):
            @plsc.parallel_loop(lo, hi, unroll=1)
            def lin(g):
                cj = colsx[pl.ds(q * 4 * BLK + 1 * BLK + g * 16, 16)]
                rg = colsx[pl.ds(q * 4 * BLK + 2 * BLK + g * 16, 16)]
                rb = p * BLK + g * 16
                for k in range(16):
                    cjk = cj[k]
                    rgk = rg[k]
                    for c in range(NCH):
                        plsc.addupdate(
                            rows2.at[rb + k, pl.ds(c * LANES, LANES)],
                            cjk * w0[c] + rgk * w1[c],
                        )

        gather_wait_n(3)
        lin_part(0, 15)

        gather_wait_n(2)

        @pl.when(j >= 1)
        def _():
            out_wait()

        @pl.when(j < NBLK - 1)
        def _():
            gather_start(j + 1)

        lin_part(15, BLK // 16)

        out_start(j)
        return 0

    lax.fori_loop(0, NBLK, block_body, 0)
    out_wait()


@jax.jit
def _run(feat, bond, st, wt, b):
    mesh = plsc.VectorSubcoreMesh(core_axis_name="c", subcore_axis_name="s")
    return pl.kernel(
        _body,
        out_type=jax.ShapeDtypeStruct((E, D), jnp.float32),
        mesh=mesh,
        scratch_types=[
            pltpu.VMEM((3 * 4 * BLK,), jnp.float32),
            pltpu.VMEM((2 * NSUB, SUB), jnp.int32),
            pltpu.VMEM((2 * BLK, D), jnp.float32),
            pltpu.VMEM((10 * D,), jnp.float32),
            pltpu.VMEM((7 * D,), jnp.float32),
            pltpu.VMEM((2 * D,), jnp.float32),
            pltpu.VMEM((D,), jnp.float32),
            pltpu.VMEM((NTAB, D), jnp.float32),
            pltpu.VMEM_SHARED((16 * NTAB, D), jnp.float32),
            pltpu.SemaphoreType.DMA,
            pltpu.SemaphoreType.DMA,
            pltpu.SemaphoreType.DMA,
        ],
    )(feat, bond, st, wt, b)


def kernel(edge_features, bond_type_table, stereo_table, W_binary, b_binary):
    feat = edge_features.T.reshape(-1)
    bond = bond_type_table.reshape(-1)
    st = stereo_table.reshape(-1)
    wt = W_binary.T.reshape(-1)
    return _run(feat, bond, st, wt, b_binary)


# speedup vs baseline: 1.1325x; 1.0050x over previous
import jax
import jax.numpy as jnp
from jax import lax
from jax.experimental import pallas as pl
from jax.experimental.pallas import tpu as pltpu
from jax.experimental.pallas import tpu_sc as plsc

E = 320000
D = 128
LANES = 16
NCH = D // LANES
NW = 32
EPW = E // NW
BLK = 400
NBLK = EPW // BLK
SUB = 80
NSUB = BLK // SUB
NTAB = 70


def _body(feat_hbm, bond_hbm, st_hbm, wt_hbm, b_hbm, out_hbm,
          colsx, idx2, rows2, bond_v, st_v, wt_v, b_v,
          tab_v, tab_sh, gsem, fsem, osem):
    sid = lax.axis_index("s")
    wid = sid * 2 + lax.axis_index("c")
    base = wid * EPW

    pltpu.sync_copy(bond_hbm, bond_v)
    pltpu.sync_copy(st_hbm, st_v)
    pltpu.sync_copy(wt_hbm, wt_v)
    pltpu.sync_copy(b_hbm, b_v)

    def build_row(i, _):
        bi = i // 7
        si = i - bi * 7
        for c in range(NCH):
            tab_v[i, pl.ds(c * LANES, LANES)] = (
                bond_v[pl.ds(bi * D + c * LANES, LANES)]
                + st_v[pl.ds(si * D + c * LANES, LANES)]
                + b_v[pl.ds(c * LANES, LANES)]
            )
        return 0

    lax.fori_loop(0, NTAB, build_row, 0)

    pltpu.sync_copy(tab_v, tab_sh.at[pl.ds(sid * NTAB, NTAB), :])

    w0 = [wt_v[pl.ds(c * LANES, LANES)] for c in range(NCH)]
    w1 = [wt_v[pl.ds(D + c * LANES, LANES)] for c in range(NCH)]

    FB4 = 4 * BLK

    def feat_start(j):
        eb = base + j * BLK
        q = j - (j // 3) * 3
        for c in range(4):
            pltpu.async_copy(feat_hbm.at[pl.ds(c * E + eb, BLK)],
                             colsx.at[pl.ds(q * FB4 + c * BLK, BLK)], fsem)

    def feat_wait():
        pltpu.make_async_copy(feat_hbm.at[pl.ds(0, FB4)],
                              colsx.at[pl.ds(0, FB4)], fsem).wait()

    def idx_compute(j):
        p = j & 1
        qj = j - (j // 3) * 3

        @plsc.parallel_loop(0, BLK // 16, unroll=2)
        def grp(g):
            f0 = colsx[pl.ds(qj * 4 * BLK + 0 * BLK + g * 16, 16)]
            f3 = colsx[pl.ds(qj * 4 * BLK + 3 * BLK + g * 16, 16)]
            bond = jnp.clip((f0 * 2.0).astype(jnp.int32), 0, 9)
            st = jnp.clip(f3.astype(jnp.int32), 0, 6)
            r = g // (SUB // 16)
            co = (g - r * (SUB // 16)) * 16
            idx2[p * NSUB + r, pl.ds(co, 16)] = bond * 7 + st + sid * NTAB

    def gather_start(j):
        p = j & 1
        for i in range(NSUB):
            pltpu.async_copy(tab_sh.at[idx2.at[p * NSUB + i]],
                             rows2.at[pl.ds(p * BLK + i * SUB, SUB), :], gsem)

    def gather_wait_n(k):
        pltpu.make_async_copy(tab_sh.at[idx2.at[0]],
                              rows2.at[pl.ds(0, k * SUB), :], gsem).wait()

    def out_start(j):
        p = j & 1
        eb = base + j * BLK
        pltpu.async_copy(rows2.at[pl.ds(p * BLK, BLK), :],
                         out_hbm.at[pl.ds(eb, BLK), :], osem)

    def out_wait():
        pltpu.make_async_copy(rows2.at[pl.ds(0, BLK), :],
                              out_hbm.at[pl.ds(base, BLK), :], osem).wait()

    feat_start(0)
    feat_start(1)
    feat_wait()
    idx_compute(0)
    gather_start(0)

    def block_body(j, _):
        p = j & 1
        q = j - (j // 3) * 3

        @pl.when(j < NBLK - 2)
        def _():
            feat_start(j + 2)

        @pl.when(j < NBLK - 1)
        def _():
            feat_wait()
            idx_compute(j + 1)

        def lin_part(lo, hi):
            @plsc.parallel_loop(lo, hi, unroll=1)
            def lin(g):
                cj = colsx[pl.ds(q * 4 * BLK + 1 * BLK + g * 16, 16)]
                rg = colsx[pl.ds(q * 4 * BLK + 2 * BLK + g * 16, 16)]
                rb = p * BLK + g * 16
                for k in range(16):
                    cjk = cj[k]
                    rgk = rg[k]
                    for c in range(NCH):
                        plsc.addupdate(
                            rows2.at[rb + k, pl.ds(c * LANES, LANES)],
                            cjk * w0[c] + rgk * w1[c],
                        )

        gather_wait_n(3)
        lin_part(0, 15)

        gather_wait_n(2)

        @pl.when(j >= 1)
        def _():
            out_wait()

        @pl.when(j < NBLK - 1)
        def _():
            gather_start(j + 1)

        lin_part(15, BLK // 16)

        out_start(j)
        return 0

    lax.fori_loop(0, NBLK, block_body, 0)
    out_wait()


@jax.jit
def _run(feat, bond, st, wt, b):
    mesh = plsc.VectorSubcoreMesh(core_axis_name="c", subcore_axis_name="s")
    return pl.kernel(
        _body,
        out_type=jax.ShapeDtypeStruct((E, D), jnp.float32),
        mesh=mesh,
        scratch_types=[
            pltpu.VMEM((3 * 4 * BLK,), jnp.float32),
            pltpu.VMEM((2 * NSUB, SUB), jnp.int32),
            pltpu.VMEM((2 * BLK, D), jnp.float32),
            pltpu.VMEM((10 * D,), jnp.float32),
            pltpu.VMEM((7 * D,), jnp.float32),
            pltpu.VMEM((2 * D,), jnp.float32),
            pltpu.VMEM((D,), jnp.float32),
            pltpu.VMEM((NTAB, D), jnp.float32),
            pltpu.VMEM_SHARED((16 * NTAB, D), jnp.float32),
            pltpu.SemaphoreType.DMA,
            pltpu.SemaphoreType.DMA,
            pltpu.SemaphoreType.DMA,
        ],
    )(feat, bond, st, wt, b)


def kernel(edge_features, bond_type_table, stereo_table, W_binary, b_binary):
    feat = edge_features.T.reshape(-1)
    bond = bond_type_table.reshape(-1)
    st = stereo_table.reshape(-1)
    wt = W_binary.T.reshape(-1)
    return _run(feat, bond, st, wt, b_binary)
